# baseline (speedup 1.0000x reference)
import functools

import jax
import jax.numpy as jnp
from jax import lax
from jax.experimental import pallas as pl
from jax.experimental.pallas import tpu as pltpu
from jax.experimental.pallas import tpu_sc as plsc

N = 10000
D = 128
DH = D // 2
E = 320000

NC = 2
NS = 16
CHUNK = 128
CPT = 160
EPT = CPT * CHUNK
E_PAD = NS * EPT
N_PAD = 10240
RPT = N_PAD // NS
KPT = RPT // CHUNK
HPT = N // NS
TRASH = N
CW = 16
NBUF = 4
DEPTH = NBUF // 2
NIDX = 8

_MESH = plsc.VectorSubcoreMesh(core_axis_name="c", subcore_axis_name="s")


def _seg_sum_body(with_cnt, hs_hbm, idx_hbm, *rest):
    if with_cnt:
        (out_p, out_c, sidx, rows, sem_i, sem_g, sem_s, acc, h_sp,
         ones_v, sem_c, accc) = rest
    else:
        out_p, sidx, rows, sem_i, sem_g, sem_s, acc, h_sp = rest
    c = lax.axis_index("c")
    s = lax.axis_index("s")

    pltpu.sync_copy(hs_hbm.at[c].at[pl.ds(s * HPT, HPT)],
                    h_sp.at[pl.ds(s * HPT, HPT)])

    @pl.loop(0, CHUNK)
    def _(r):
        @pl.loop(0, DH, step=16)
        def _(cc):
            rows.at[0, r, pl.ds(cc, 16)][...] = jnp.zeros((16,), jnp.float32)

    for k in range(KPT):
        pltpu.sync_copy(rows.at[0],
                        acc.at[pl.ds(s * RPT + k * CHUNK, CHUNK)])

    if with_cnt:
        @pl.loop(0, CHUNK)
        def _(r):
            ones_v.at[r, pl.ds(0, CW)][...] = jnp.zeros((CW,), jnp.float32)

        for k in range(KPT):
            pltpu.sync_copy(
                ones_v, accc.at[pl.ds(s * RPT + k * CHUNK, CHUNK)])

        @pl.loop(0, CHUNK)
        def _(r):
            ones_v.at[r, pl.ds(0, CW)][...] = jnp.ones((CW,), jnp.float32)

    plsc.subcore_barrier()

    def idx_start(j, ib):
        pltpu.async_copy(idx_hbm.at[s].at[j], sidx.at[ib], sem_i)

    def idx_wait(j, ib):
        pltpu.make_async_copy(idx_hbm.at[s].at[j], sidx.at[ib], sem_i).wait()

    def gather_start(ib, rb):
        pltpu.async_copy(h_sp.at[sidx.at[ib].at[0]], rows.at[rb], sem_g)

    def gather_wait(ib, rb):
        pltpu.make_async_copy(
            h_sp.at[sidx.at[ib].at[0]], rows.at[rb], sem_g).wait()

    def scat_start(ib, rb):
        pltpu.async_copy(rows.at[rb], acc.at[sidx.at[ib].at[1]], sem_s,
                         add=True)

    def scat_wait(ib, rb):
        pltpu.make_async_copy(
            rows.at[rb], acc.at[sidx.at[ib].at[1]], sem_s).wait()

    def cnt_start(ib):
        pltpu.async_copy(ones_v, accc.at[sidx.at[ib].at[1]], sem_c, add=True)

    def cnt_wait(ib):
        pltpu.make_async_copy(ones_v, accc.at[sidx.at[ib].at[1]], sem_c).wait()

    for t in range(DEPTH + 1):
        idx_start(t, t % NIDX)
    for b in range(DEPTH):
        idx_wait(b, b % NIDX)
        gather_start(b % NIDX, b % NBUF)

    @pl.loop(0, CPT, step=NIDX)
    def _(g):
        for b in range(NIDX):
            j = g + b
            gather_wait(b % NIDX, b % NBUF)
            scat_start(b % NIDX, b % NBUF)
            if with_cnt:
                @pl.when(c == b % 2)
                def _():
                    cnt_start(b % NIDX)

                @pl.when(jnp.logical_and(c == b % 2, j >= DEPTH))
                def _():
                    cnt_wait(b % NIDX)

            @pl.when(j >= DEPTH)
            def _():
                scat_wait((b - DEPTH) % NIDX, (b - DEPTH) % NBUF)

            @pl.when(j + DEPTH < CPT)
            def _():
                idx_wait(j + DEPTH, (b + DEPTH) % NIDX)
                gather_start((b + DEPTH) % NIDX, (b + DEPTH) % NBUF)

            @pl.when(j + DEPTH + 1 < CPT)
            def _():
                idx_start(j + DEPTH + 1, (b + DEPTH + 1) % NIDX)

    for b in range(DEPTH):
        j = CPT - DEPTH + b
        scat_wait(j % NIDX, j % NBUF)
        if with_cnt:
            @pl.when(c == j % 2)
            def _():
                cnt_wait(j % NIDX)

    plsc.subcore_barrier()

    pltpu.sync_copy(acc.at[pl.ds(s * RPT, RPT)],
                    out_p.at[c].at[pl.ds(s * RPT, RPT)])
    if with_cnt:
        pltpu.sync_copy(accc.at[pl.ds(s * RPT, RPT)],
                        out_c.at[c].at[pl.ds(s * RPT, RPT)])


def _seg_sum(hs, idx, with_cnt):
    outs = [jax.ShapeDtypeStruct((NC, N_PAD, DH), jnp.float32)]
    scratch = [
        pltpu.VMEM((NIDX, 2, CHUNK), jnp.int32),
        pltpu.VMEM((NBUF, CHUNK, DH), jnp.float32),
        pltpu.SemaphoreType.DMA,
        pltpu.SemaphoreType.DMA,
        pltpu.SemaphoreType.DMA,
        pltpu.VMEM_SHARED((N_PAD, DH), jnp.float32),
        pltpu.VMEM_SHARED((N, DH), jnp.float32),
    ]
    if with_cnt:
        outs.append(jax.ShapeDtypeStruct((NC, N_PAD, CW), jnp.float32))
        scratch += [
            pltpu.VMEM((CHUNK, CW), jnp.float32),
            pltpu.SemaphoreType.DMA,
            pltpu.VMEM_SHARED((N_PAD, CW), jnp.float32),
        ]
    fn = pl.kernel(
        functools.partial(_seg_sum_body, with_cnt),
        out_type=tuple(outs),
        mesh=_MESH,
        scratch_types=scratch,
        compiler_params=pltpu.CompilerParams(use_tc_tiling_on_sc=False),
    )
    return fn(hs, idx)


_RB = 2000
_GRID = N // _RB


def _linear_body(x_ref, w_ref, b_ref, o_ref):
    z = lax.dot_general(
        x_ref[...], w_ref[...], (((1,), (1,)), ((), ())),
        preferred_element_type=jnp.float32) + b_ref[...]
    o_ref[...] = jnp.stack([z[:, :DH], z[:, DH:]])


def _linear_split(x, w, b):
    return pl.pallas_call(
        _linear_body,
        grid=(_GRID,),
        in_specs=[
            pl.BlockSpec((_RB, D), lambda i: (i, 0)),
            pl.BlockSpec((D, D), lambda i: (0, 0)),
            pl.BlockSpec((1, D), lambda i: (0, 0)),
        ],
        out_specs=pl.BlockSpec((NC, _RB, DH), lambda i: (0, i, 0)),
        out_shape=jax.ShapeDtypeStruct((NC, N, DH), jnp.float32),
    )(x, w, b.reshape(1, D))


def _self_body(h_ref, wr_ref, b_ref, o_ref):
    h = jnp.concatenate([h_ref[0], h_ref[1]], axis=1)
    o_ref[...] = lax.dot_general(
        h, wr_ref[...], (((1,), (1,)), ((), ())),
        preferred_element_type=jnp.float32) + b_ref[...]


def _self_tc(hs, wr, b):
    return pl.pallas_call(
        _self_body,
        grid=(_GRID,),
        in_specs=[
            pl.BlockSpec((NC, _RB, DH), lambda i: (0, i, 0)),
            pl.BlockSpec((D, D), lambda i: (0, 0)),
            pl.BlockSpec((1, D), lambda i: (0, 0)),
        ],
        out_specs=pl.BlockSpec((_RB, D), lambda i: (i, 0)),
        out_shape=jax.ShapeDtypeStruct((N, D), jnp.float32),
    )(hs, wr, b.reshape(1, D))


def _comb_body(mode, p_ref, c_ref, s_ref, wl_ref, o_ref):
    cnt = jnp.maximum(c_ref[0][:, 0:1] + c_ref[1][:, 0:1], 1.0)
    mean = jnp.concatenate([p_ref[0], p_ref[1]], axis=1) / cnt
    z = lax.dot_general(mean, wl_ref[...], (((1,), (1,)), ((), ())),
                        preferred_element_type=jnp.float32) + s_ref[...]
    if mode == "relu":
        z = jnp.maximum(z, 0.0)
        o_ref[...] = jnp.stack([z[:, :DH], z[:, DH:]])
    else:
        nrm = jnp.sqrt(jnp.sum(z * z, axis=1, keepdims=True))
        o_ref[...] = z / jnp.maximum(nrm, 1e-12)


def _comb_tc(p, cnt, selfz, wl, mode):
    if mode == "relu":
        out_spec = pl.BlockSpec((NC, _RB, DH), lambda i: (0, i, 0))
        out_shape = jax.ShapeDtypeStruct((NC, N, DH), jnp.float32)
    else:
        out_spec = pl.BlockSpec((_RB, D), lambda i: (i, 0))
        out_shape = jax.ShapeDtypeStruct((N, D), jnp.float32)
    return pl.pallas_call(
        functools.partial(_comb_body, mode),
        grid=(_GRID,),
        in_specs=[
            pl.BlockSpec((NC, _RB, DH), lambda i: (0, i, 0)),
            pl.BlockSpec((NC, _RB, CW), lambda i: (0, i, 0)),
            pl.BlockSpec((_RB, D), lambda i: (i, 0)),
            pl.BlockSpec((D, D), lambda i: (0, 0)),
        ],
        out_specs=out_spec,
        out_shape=out_shape,
    )(p, cnt, selfz, wl)


def kernel(x, edge_index, W_pre, b_pre, Wl1, bl1, Wr1, Wl2, bl2, Wr2,
           Wl3, bl3, Wr3):
    src = edge_index[0].astype(jnp.int32)
    dst = edge_index[1].astype(jnp.int32)
    pad = E_PAD - E
    sp3 = jnp.concatenate(
        [src, jnp.zeros((pad,), jnp.int32)]).reshape(NS, CPT, CHUNK)
    dst3 = jnp.concatenate(
        [dst, jnp.full((pad,), TRASH, jnp.int32)]).reshape(NS, CPT, CHUNK)
    idx = jnp.stack([sp3, dst3], axis=2)

    h0s = _linear_split(x, W_pre, b_pre)
    s1 = _self_tc(h0s, Wr1, bl1)
    p1, cnt = _seg_sum(h0s, idx, with_cnt=True)
    h1s = _comb_tc(p1, cnt, s1, Wl1, "relu")
    s2 = _self_tc(h1s, Wr2, bl2)
    p2, = _seg_sum(h1s, idx, with_cnt=False)
    h2s = _comb_tc(p2, cnt, s2, Wl2, "relu")
    s3 = _self_tc(h2s, Wr3, bl3)
    p3, = _seg_sum(h2s, idx, with_cnt=False)
    return _comb_tc(p3, cnt, s3, Wl3, "norm")


# speedup vs baseline: 1.0280x; 1.0280x over previous
import functools

import jax
import jax.numpy as jnp
from jax import lax
from jax.experimental import pallas as pl
from jax.experimental.pallas import tpu as pltpu
from jax.experimental.pallas import tpu_sc as plsc

N = 10000
D = 128
DH = D // 2
E = 320000

NC = 2
NS = 16
CHUNK = 128
CPT = 160
EPT = CPT * CHUNK
E_PAD = NS * EPT
N_PAD = 10240
RPT = N_PAD // NS
KPT = RPT // CHUNK
HPT = N // NS
TRASH = N
CW = 8
NBUF = 4
DEPTH = NBUF // 2
NIDX = 8

_MESH = plsc.VectorSubcoreMesh(core_axis_name="c", subcore_axis_name="s")


def _seg_sum_body(with_cnt, hs_hbm, idx_hbm, *rest):
    if with_cnt:
        (out_p, out_c, sidx, rows, sem_i, sem_g, sem_s, acc, h_sp,
         ones_v, sem_c, accc) = rest
    else:
        out_p, sidx, rows, sem_i, sem_g, sem_s, acc, h_sp = rest
    c = lax.axis_index("c")
    s = lax.axis_index("s")

    pltpu.sync_copy(hs_hbm.at[c].at[pl.ds(s * HPT, HPT)],
                    h_sp.at[pl.ds(s * HPT, HPT)])

    @pl.loop(0, CHUNK)
    def _(r):
        @pl.loop(0, DH, step=16)
        def _(cc):
            rows.at[0, r, pl.ds(cc, 16)][...] = jnp.zeros((16,), jnp.float32)

    for k in range(KPT):
        pltpu.sync_copy(rows.at[0],
                        acc.at[pl.ds(s * RPT + k * CHUNK, CHUNK)])

    if with_cnt:
        @pl.loop(0, CHUNK)
        def _(r):
            ones_v.at[r, pl.ds(0, CW)][...] = jnp.zeros((CW,), jnp.float32)

        for k in range(KPT):
            pltpu.sync_copy(
                ones_v, accc.at[pl.ds(s * RPT + k * CHUNK, CHUNK)])

        @pl.loop(0, CHUNK)
        def _(r):
            ones_v.at[r, pl.ds(0, CW)][...] = jnp.ones((CW,), jnp.float32)

    plsc.subcore_barrier()

    def idx_start(j, ib):
        pltpu.async_copy(idx_hbm.at[s].at[j], sidx.at[ib], sem_i)

    def idx_wait(j, ib):
        pltpu.make_async_copy(idx_hbm.at[s].at[j], sidx.at[ib], sem_i).wait()

    def gather_start(ib, rb):
        pltpu.async_copy(h_sp.at[sidx.at[ib].at[0]], rows.at[rb], sem_g)

    def gather_wait(ib, rb):
        pltpu.make_async_copy(
            h_sp.at[sidx.at[ib].at[0]], rows.at[rb], sem_g).wait()

    def scat_start(ib, rb):
        pltpu.async_copy(rows.at[rb], acc.at[sidx.at[ib].at[1]], sem_s,
                         add=True)

    def scat_wait(ib, rb):
        pltpu.make_async_copy(
            rows.at[rb], acc.at[sidx.at[ib].at[1]], sem_s).wait()

    def cnt_start(ib):
        pltpu.async_copy(ones_v, accc.at[sidx.at[ib].at[1]], sem_c, add=True)

    def cnt_wait(ib):
        pltpu.make_async_copy(ones_v, accc.at[sidx.at[ib].at[1]], sem_c).wait()

    for t in range(DEPTH + 1):
        idx_start(t, t % NIDX)
    for b in range(DEPTH):
        idx_wait(b, b % NIDX)
        gather_start(b % NIDX, b % NBUF)

    @pl.loop(0, CPT, step=NIDX)
    def _(g):
        for b in range(NIDX):
            j = g + b
            gather_wait(b % NIDX, b % NBUF)
            scat_start(b % NIDX, b % NBUF)
            if with_cnt:
                @pl.when(c == b % 2)
                def _():
                    cnt_start(b % NIDX)

                @pl.when(jnp.logical_and(c == b % 2, j >= DEPTH))
                def _():
                    cnt_wait(b % NIDX)

            @pl.when(j >= DEPTH)
            def _():
                scat_wait((b - DEPTH) % NIDX, (b - DEPTH) % NBUF)

            @pl.when(j + DEPTH < CPT)
            def _():
                idx_wait(j + DEPTH, (b + DEPTH) % NIDX)
                gather_start((b + DEPTH) % NIDX, (b + DEPTH) % NBUF)

            @pl.when(j + DEPTH + 1 < CPT)
            def _():
                idx_start(j + DEPTH + 1, (b + DEPTH + 1) % NIDX)

    for b in range(DEPTH):
        j = CPT - DEPTH + b
        scat_wait(j % NIDX, j % NBUF)
        if with_cnt:
            @pl.when(c == j % 2)
            def _():
                cnt_wait(j % NIDX)

    plsc.subcore_barrier()

    pltpu.sync_copy(acc.at[pl.ds(s * RPT, RPT)],
                    out_p.at[c].at[pl.ds(s * RPT, RPT)])
    if with_cnt:
        pltpu.sync_copy(accc.at[pl.ds(s * RPT, RPT)],
                        out_c.at[c].at[pl.ds(s * RPT, RPT)])


def _seg_sum(hs, idx, with_cnt):
    outs = [jax.ShapeDtypeStruct((NC, N_PAD, DH), jnp.float32)]
    scratch = [
        pltpu.VMEM((NIDX, 2, CHUNK), jnp.int32),
        pltpu.VMEM((NBUF, CHUNK, DH), jnp.float32),
        pltpu.SemaphoreType.DMA,
        pltpu.SemaphoreType.DMA,
        pltpu.SemaphoreType.DMA,
        pltpu.VMEM_SHARED((N_PAD, DH), jnp.float32),
        pltpu.VMEM_SHARED((N, DH), jnp.float32),
    ]
    if with_cnt:
        outs.append(jax.ShapeDtypeStruct((NC, N_PAD, CW), jnp.float32))
        scratch += [
            pltpu.VMEM((CHUNK, CW), jnp.float32),
            pltpu.SemaphoreType.DMA,
            pltpu.VMEM_SHARED((N_PAD, CW), jnp.float32),
        ]
    fn = pl.kernel(
        functools.partial(_seg_sum_body, with_cnt),
        out_type=tuple(outs),
        mesh=_MESH,
        scratch_types=scratch,
        compiler_params=pltpu.CompilerParams(use_tc_tiling_on_sc=False),
    )
    return fn(hs, idx)


_RB = 2000
_GRID = N // _RB


def _linear_body(x_ref, w_ref, b_ref, o_ref):
    z = lax.dot_general(
        x_ref[...], w_ref[...], (((1,), (1,)), ((), ())),
        preferred_element_type=jnp.float32) + b_ref[...]
    o_ref[...] = jnp.stack([z[:, :DH], z[:, DH:]])


def _linear_split(x, w, b):
    return pl.pallas_call(
        _linear_body,
        grid=(_GRID,),
        in_specs=[
            pl.BlockSpec((_RB, D), lambda i: (i, 0)),
            pl.BlockSpec((D, D), lambda i: (0, 0)),
            pl.BlockSpec((1, D), lambda i: (0, 0)),
        ],
        out_specs=pl.BlockSpec((NC, _RB, DH), lambda i: (0, i, 0)),
        out_shape=jax.ShapeDtypeStruct((NC, N, DH), jnp.float32),
    )(x, w, b.reshape(1, D))


def _sage_body(mode, p_ref, c_ref, h_ref, wl_ref, bl_ref, wr_ref, o_ref):
    cnt = jnp.maximum(c_ref[0][:, 0:1] + c_ref[1][:, 0:1], 1.0)
    mean = jnp.concatenate([p_ref[0], p_ref[1]], axis=1) / cnt
    h = jnp.concatenate([h_ref[0], h_ref[1]], axis=1)
    z = lax.dot_general(mean, wl_ref[...], (((1,), (1,)), ((), ())),
                        preferred_element_type=jnp.float32)
    z = z + lax.dot_general(h, wr_ref[...], (((1,), (1,)), ((), ())),
                            preferred_element_type=jnp.float32)
    z = z + bl_ref[...]
    if mode == "relu":
        z = jnp.maximum(z, 0.0)
        o_ref[...] = jnp.stack([z[:, :DH], z[:, DH:]])
    else:
        nrm = jnp.sqrt(jnp.sum(z * z, axis=1, keepdims=True))
        o_ref[...] = z / jnp.maximum(nrm, 1e-12)


def _sage_tc(p, cnt, hs, wl, bl, wr, mode):
    if mode == "relu":
        out_spec = pl.BlockSpec((NC, _RB, DH), lambda i: (0, i, 0))
        out_shape = jax.ShapeDtypeStruct((NC, N, DH), jnp.float32)
    else:
        out_spec = pl.BlockSpec((_RB, D), lambda i: (i, 0))
        out_shape = jax.ShapeDtypeStruct((N, D), jnp.float32)
    return pl.pallas_call(
        functools.partial(_sage_body, mode),
        grid=(_GRID,),
        in_specs=[
            pl.BlockSpec((NC, _RB, DH), lambda i: (0, i, 0)),
            pl.BlockSpec((NC, _RB, CW), lambda i: (0, i, 0)),
            pl.BlockSpec((NC, _RB, DH), lambda i: (0, i, 0)),
            pl.BlockSpec((D, D), lambda i: (0, 0)),
            pl.BlockSpec((1, D), lambda i: (0, 0)),
            pl.BlockSpec((D, D), lambda i: (0, 0)),
        ],
        out_specs=out_spec,
        out_shape=out_shape,
    )(p, cnt, hs, wl, bl.reshape(1, D), wr)


def kernel(x, edge_index, W_pre, b_pre, Wl1, bl1, Wr1, Wl2, bl2, Wr2,
           Wl3, bl3, Wr3):
    src = edge_index[0].astype(jnp.int32)
    dst = edge_index[1].astype(jnp.int32)
    pad = E_PAD - E
    sp3 = jnp.concatenate(
        [src, jnp.zeros((pad,), jnp.int32)]).reshape(NS, CPT, CHUNK)
    dst3 = jnp.concatenate(
        [dst, jnp.full((pad,), TRASH, jnp.int32)]).reshape(NS, CPT, CHUNK)
    idx = jnp.stack([sp3, dst3], axis=2)

    h0s = _linear_split(x, W_pre, b_pre)
    p1, cnt = _seg_sum(h0s, idx, with_cnt=True)
    h1s = _sage_tc(p1, cnt, h0s, Wl1, bl1, Wr1, "relu")
    p2, = _seg_sum(h1s, idx, with_cnt=False)
    h2s = _sage_tc(p2, cnt, h1s, Wl2, bl2, Wr2, "relu")
    p3, = _seg_sum(h2s, idx, with_cnt=False)
    return _sage_tc(p3, cnt, h2s, Wl3, bl3, Wr3, "norm")


# speedup vs baseline: 1.0324x; 1.0043x over previous
import functools

import jax
import jax.numpy as jnp
from jax import lax
from jax.experimental import pallas as pl
from jax.experimental.pallas import tpu as pltpu
from jax.experimental.pallas import tpu_sc as plsc

N = 10000
D = 128
DH = D // 2
E = 320000

NC = 2
NS = 16
CHUNK = 128
CPT = 160
EPT = CPT * CHUNK
E_PAD = NS * EPT
N_PAD = 10240
RPT = N_PAD // NS
KPT = RPT // CHUNK
HPT = N // NS
TRASH = N
CW = 8
NBUF = 4
DEPTH = NBUF // 2
NIDX = 8

_MESH = plsc.VectorSubcoreMesh(core_axis_name="c", subcore_axis_name="s")


def _seg_sum_body(with_cnt, hs_hbm, idx_hbm, *rest):
    if with_cnt:
        (out_p, out_c, sidx, rows, sem_i, sem_g, sem_s, acc, h_sp,
         ones_v, sem_c, accc) = rest
    else:
        out_p, sidx, rows, sem_i, sem_g, sem_s, acc, h_sp = rest
    c = lax.axis_index("c")
    s = lax.axis_index("s")

    pltpu.sync_copy(hs_hbm.at[c].at[pl.ds(s * HPT, HPT)],
                    h_sp.at[pl.ds(s * HPT, HPT)])

    @pl.loop(0, CHUNK)
    def _(r):
        @pl.loop(0, DH, step=16)
        def _(cc):
            rows.at[0, r, pl.ds(cc, 16)][...] = jnp.zeros((16,), jnp.float32)

    for k in range(KPT):
        pltpu.sync_copy(rows.at[0],
                        acc.at[pl.ds(s * RPT + k * CHUNK, CHUNK)])

    if with_cnt:
        @pl.loop(0, CHUNK)
        def _(r):
            ones_v.at[r, pl.ds(0, CW)][...] = jnp.zeros((CW,), jnp.float32)

        for k in range(KPT):
            pltpu.sync_copy(
                ones_v, accc.at[pl.ds(s * RPT + k * CHUNK, CHUNK)])

        @pl.loop(0, CHUNK)
        def _(r):
            ones_v.at[r, pl.ds(0, CW)][...] = jnp.ones((CW,), jnp.float32)

    plsc.subcore_barrier()

    def idx_start(j, ib):
        pltpu.async_copy(idx_hbm.at[s].at[j], sidx.at[ib], sem_i)

    def idx_wait(j, ib):
        pltpu.make_async_copy(idx_hbm.at[s].at[j], sidx.at[ib], sem_i).wait()

    def gather_start(ib, rb):
        pltpu.async_copy(h_sp.at[sidx.at[ib].at[0]], rows.at[rb], sem_g)

    def gather_wait(ib, rb):
        pltpu.make_async_copy(
            h_sp.at[sidx.at[ib].at[0]], rows.at[rb], sem_g).wait()

    def scat_start(ib, rb):
        pltpu.async_copy(rows.at[rb], acc.at[sidx.at[ib].at[1]], sem_s,
                         add=True)

    def scat_wait(ib, rb):
        pltpu.make_async_copy(
            rows.at[rb], acc.at[sidx.at[ib].at[1]], sem_s).wait()

    def cnt_start(ib):
        pltpu.async_copy(ones_v, accc.at[sidx.at[ib].at[1]], sem_c, add=True)

    def cnt_wait(ib):
        pltpu.make_async_copy(ones_v, accc.at[sidx.at[ib].at[1]], sem_c).wait()

    for t in range(DEPTH + 1):
        idx_start(t, t % NIDX)
    for b in range(DEPTH):
        idx_wait(b, b % NIDX)
        gather_start(b % NIDX, b % NBUF)

    @pl.loop(0, CPT, step=NIDX)
    def _(g):
        for b in range(NIDX):
            j = g + b
            gather_wait(b % NIDX, b % NBUF)
            scat_start(b % NIDX, b % NBUF)
            if with_cnt:
                @pl.when(c == b % 2)
                def _():
                    cnt_start(b % NIDX)

                @pl.when(jnp.logical_and(c == b % 2, j >= DEPTH))
                def _():
                    cnt_wait(b % NIDX)

            @pl.when(j >= DEPTH)
            def _():
                scat_wait((b - DEPTH) % NIDX, (b - DEPTH) % NBUF)

            @pl.when(j + DEPTH < CPT)
            def _():
                idx_wait(j + DEPTH, (b + DEPTH) % NIDX)
                gather_start((b + DEPTH) % NIDX, (b + DEPTH) % NBUF)

            @pl.when(j + DEPTH + 1 < CPT)
            def _():
                idx_start(j + DEPTH + 1, (b + DEPTH + 1) % NIDX)

    for b in range(DEPTH):
        j = CPT - DEPTH + b
        scat_wait(j % NIDX, j % NBUF)
        if with_cnt:
            @pl.when(c == j % 2)
            def _():
                cnt_wait(j % NIDX)

    plsc.subcore_barrier()

    pltpu.sync_copy(acc.at[pl.ds(s * RPT, RPT)],
                    out_p.at[c].at[pl.ds(s * RPT, RPT)])
    if with_cnt:
        pltpu.sync_copy(accc.at[pl.ds(s * RPT, RPT)],
                        out_c.at[c].at[pl.ds(s * RPT, RPT)])


def _seg_sum(hs, idx, with_cnt):
    outs = [jax.ShapeDtypeStruct((NC, N_PAD, DH), jnp.float32)]
    scratch = [
        pltpu.VMEM((NIDX, 2, CHUNK), jnp.int32),
        pltpu.VMEM((NBUF, CHUNK, DH), jnp.float32),
        pltpu.SemaphoreType.DMA,
        pltpu.SemaphoreType.DMA,
        pltpu.SemaphoreType.DMA,
        pltpu.VMEM_SHARED((N_PAD, DH), jnp.float32),
        pltpu.VMEM_SHARED((N, DH), jnp.float32),
    ]
    if with_cnt:
        outs.append(jax.ShapeDtypeStruct((NC, N_PAD, CW), jnp.float32))
        scratch += [
            pltpu.VMEM((CHUNK, CW), jnp.float32),
            pltpu.SemaphoreType.DMA,
            pltpu.VMEM_SHARED((N_PAD, CW), jnp.float32),
        ]
    fn = pl.kernel(
        functools.partial(_seg_sum_body, with_cnt),
        out_type=tuple(outs),
        mesh=_MESH,
        scratch_types=scratch,
        compiler_params=pltpu.CompilerParams(use_tc_tiling_on_sc=False),
    )
    return fn(hs, idx)


_RB = 5000
_GRID = N // _RB


def _linear_body(x_ref, w_ref, b_ref, o_ref):
    z = lax.dot_general(
        x_ref[...], w_ref[...], (((1,), (1,)), ((), ())),
        preferred_element_type=jnp.float32) + b_ref[...]
    o_ref[...] = jnp.stack([z[:, :DH], z[:, DH:]])


def _linear_split(x, w, b):
    return pl.pallas_call(
        _linear_body,
        grid=(_GRID,),
        in_specs=[
            pl.BlockSpec((_RB, D), lambda i: (i, 0)),
            pl.BlockSpec((D, D), lambda i: (0, 0)),
            pl.BlockSpec((1, D), lambda i: (0, 0)),
        ],
        out_specs=pl.BlockSpec((NC, _RB, DH), lambda i: (0, i, 0)),
        out_shape=jax.ShapeDtypeStruct((NC, N, DH), jnp.float32),
    )(x, w, b.reshape(1, D))


def _sage_body(mode, p_ref, c_ref, h_ref, wl_ref, bl_ref, wr_ref, o_ref):
    cnt = jnp.maximum(c_ref[0][:, 0:1] + c_ref[1][:, 0:1], 1.0)
    mean = jnp.concatenate([p_ref[0], p_ref[1]], axis=1) / cnt
    h = jnp.concatenate([h_ref[0], h_ref[1]], axis=1)
    z = lax.dot_general(mean, wl_ref[...], (((1,), (1,)), ((), ())),
                        preferred_element_type=jnp.float32)
    z = z + lax.dot_general(h, wr_ref[...], (((1,), (1,)), ((), ())),
                            preferred_element_type=jnp.float32)
    z = z + bl_ref[...]
    if mode == "relu":
        z = jnp.maximum(z, 0.0)
        o_ref[...] = jnp.stack([z[:, :DH], z[:, DH:]])
    else:
        nrm = jnp.sqrt(jnp.sum(z * z, axis=1, keepdims=True))
        o_ref[...] = z / jnp.maximum(nrm, 1e-12)


def _sage_tc(p, cnt, hs, wl, bl, wr, mode):
    if mode == "relu":
        out_spec = pl.BlockSpec((NC, _RB, DH), lambda i: (0, i, 0))
        out_shape = jax.ShapeDtypeStruct((NC, N, DH), jnp.float32)
    else:
        out_spec = pl.BlockSpec((_RB, D), lambda i: (i, 0))
        out_shape = jax.ShapeDtypeStruct((N, D), jnp.float32)
    return pl.pallas_call(
        functools.partial(_sage_body, mode),
        grid=(_GRID,),
        in_specs=[
            pl.BlockSpec((NC, _RB, DH), lambda i: (0, i, 0)),
            pl.BlockSpec((NC, _RB, CW), lambda i: (0, i, 0)),
            pl.BlockSpec((NC, _RB, DH), lambda i: (0, i, 0)),
            pl.BlockSpec((D, D), lambda i: (0, 0)),
            pl.BlockSpec((1, D), lambda i: (0, 0)),
            pl.BlockSpec((D, D), lambda i: (0, 0)),
        ],
        out_specs=out_spec,
        out_shape=out_shape,
    )(p, cnt, hs, wl, bl.reshape(1, D), wr)


def kernel(x, edge_index, W_pre, b_pre, Wl1, bl1, Wr1, Wl2, bl2, Wr2,
           Wl3, bl3, Wr3):
    src = edge_index[0].astype(jnp.int32)
    dst = edge_index[1].astype(jnp.int32)
    pad = E_PAD - E
    sp3 = jnp.concatenate(
        [src, jnp.zeros((pad,), jnp.int32)]).reshape(NS, CPT, CHUNK)
    dst3 = jnp.concatenate(
        [dst, jnp.full((pad,), TRASH, jnp.int32)]).reshape(NS, CPT, CHUNK)
    idx = jnp.stack([sp3, dst3], axis=2)

    h0s = _linear_split(x, W_pre, b_pre)
    p1, cnt = _seg_sum(h0s, idx, with_cnt=True)
    h1s = _sage_tc(p1, cnt, h0s, Wl1, bl1, Wr1, "relu")
    p2, = _seg_sum(h1s, idx, with_cnt=False)
    h2s = _sage_tc(p2, cnt, h1s, Wl2, bl2, Wr2, "relu")
    p3, = _seg_sum(h2s, idx, with_cnt=False)
    return _sage_tc(p3, cnt, h2s, Wl3, bl3, Wr3, "norm")
